# baseline (speedup 1.0000x reference)
import functools

import jax
import jax.numpy as jnp
from jax import lax
from jax.experimental import pallas as pl
from jax.experimental.pallas import tpu as pltpu


def _silu(x):
    return x * jax.nn.sigmoid(x)


def _fused_chain_kernel(x_ref, c0_ref, w1c0_ref, w2c0_ref, wsk0_ref, b20_ref,
                        c1_ref, w1c1_ref, w2c1_ref, b21_ref, out_ref, s_ref,
                        *, H, W):
    HW = H * W
    G = 128
    bf = jnp.bfloat16

    idx = lax.broadcasted_iota(jnp.int32, (1, HW), 1)
    col = idx % W
    m_lastcol = (col != W - 1).astype(bf)
    m_firstcol = (col != 0).astype(bf)

    def conv3x3(act_bf, wc_ref):
        C = act_bf.shape[0]
        c9 = 9 * C
        z = jnp.zeros((c9, W + 1), bf)
        s_ref[0:c9, G:G + W + 1] = z
        s_ref[0:c9, G + HW - W - 1:G + HW] = z[:, :W + 1]
        aL = act_bf * m_lastcol
        aR = act_bf * m_firstcol
        t = 0
        for dy in (-1, 0, 1):
            for dx, a in ((-1, aL), (0, act_bf), (1, aR)):
                base = G - (dy * W + dx)
                s_ref[t * C:(t + 1) * C, base:base + HW] = a
                t += 1
        return jnp.dot(wc_ref[...], s_ref[0:c9, G:G + HW],
                       preferred_element_type=jnp.float32)

    a0 = x_ref[0]

    h = conv3x3(_silu(a0.astype(jnp.float32)).astype(bf), w1c0_ref)
    c0 = c0_ref[0]
    cmid = c0.shape[0] // 2
    h = _silu(c0[:cmid] * h + c0[cmid:])
    a1 = (conv3x3(h.astype(bf), w2c0_ref) +
          jnp.dot(wsk0_ref[...], a0, preferred_element_type=jnp.float32))
    a1 = (a1 + b20_ref[...]).astype(bf)

    h = conv3x3(_silu(a1.astype(jnp.float32)).astype(bf), w1c1_ref)
    c1 = c1_ref[0]
    h = _silu(c1[:cmid] * h + c1[cmid:])
    h = conv3x3(h.astype(bf), w2c1_ref)
    out_ref[0] = a1.astype(jnp.float32) + (h + b21_ref[...])


def kernel(x, time, w1k0, b1k0, wc0, bc0, w2k0, b2k0, wskipk0,
           w1k1, b1k1, wc1, bc1, w2k1, b2k1):
    x = x.astype(jnp.float32)
    B, C0, H, W = x.shape
    HW = H * W
    bf = jnp.bfloat16
    HI = lax.Precision.HIGHEST

    c1out = w1k0.shape[1]

    def wcat(wk, cin):
        return jnp.transpose(wk, (1, 0, 2)).reshape(c1out, 9 * cin).astype(bf)

    w1c0 = wcat(w1k0, C0)
    w2c0 = wcat(w2k0, c1out)
    w1c1 = wcat(w1k1, c1out)
    w2c1 = wcat(w2k1, c1out)
    wsk0 = wskipk0.astype(bf)

    def cond_eff(wc, bc, b1):
        c = jnp.dot(time, wc, precision=HI) + bc
        scale, bias = c[:, :c1out], c[:, c1out:]
        return jnp.concatenate([scale, scale * b1.reshape(1, c1out) + bias],
                               axis=1).reshape(B, 2 * c1out, 1)

    c0 = cond_eff(wc0, bc0, b1k0)
    c1 = cond_eff(wc1, bc1, b1k1)

    def full(shape):
        n = len(shape)
        return pl.BlockSpec(shape, lambda b: (0,) * n)

    args = [x.reshape(B, C0, HW).astype(bf), c0, w1c0, w2c0, wsk0, b2k0,
            c1, w1c1, w2c1, b2k1]
    in_specs = [pl.BlockSpec((1, C0, HW), lambda b: (b, 0, 0)),
                pl.BlockSpec((1, 2 * c1out, 1), lambda b: (b, 0, 0)),
                full(w1c0.shape), full(w2c0.shape), full(wsk0.shape),
                full(b2k0.shape),
                pl.BlockSpec((1, 2 * c1out, 1), lambda b: (b, 0, 0)),
                full(w1c1.shape), full(w2c1.shape), full(b2k1.shape)]

    out = pl.pallas_call(
        functools.partial(_fused_chain_kernel, H=H, W=W),
        out_shape=jax.ShapeDtypeStruct((B, c1out, HW), jnp.float32),
        grid=(B,),
        in_specs=in_specs,
        out_specs=pl.BlockSpec((1, c1out, HW), lambda b: (b, 0, 0)),
        scratch_shapes=[pltpu.VMEM((9 * c1out, 2 * 128 + HW), bf)],
        compiler_params=pltpu.CompilerParams(
            dimension_semantics=("parallel",)),
    )(*args)
    return out.reshape(B, c1out, H, W)
